# baseline (speedup 1.0000x reference)
import functools

import jax
import jax.numpy as jnp
from jax import lax
from jax.experimental import pallas as pl
from jax.experimental.pallas import tpu as pltpu
from jax.experimental.pallas import tpu_sc as plsc

N_PEAKS_ = 64
DIM_ = 128


def _cond_body(x_ref, s_ref, b_ref, o_ref):
    s = s_ref[...][:, :, None]
    b = b_ref[...][:, :, None]
    o_ref[...] = x_ref[...] * s + b


_L = 16


def _sc_body(nrows_w, dim, t, x_hbm, ws16_hbm, wb16_hbm, out_hbm,
             in0, in1, out0, out1, s0, b0, s1, b1,
             sin0, sin1, sout0, sout1):
    nc = 2
    wid = lax.axis_index("s") * nc + lax.axis_index("c")
    row_w = dim * t
    half_w = row_w // 2
    wrow_w = dim * _L
    whalf_w = wrow_w // 2
    hd = dim // 2

    def in_descs(rl, h, ibuf, sbuf, bbuf, sem):
        row = wid * nrows_w + rl
        xoff = row * row_w + h * half_w
        woff = rl * wrow_w + h * whalf_w
        return (
            pltpu.make_async_copy(x_hbm.at[pl.ds(xoff, half_w)], ibuf, sem),
            pltpu.make_async_copy(ws16_hbm.at[pl.ds(woff, whalf_w)], sbuf, sem),
            pltpu.make_async_copy(wb16_hbm.at[pl.ds(woff, whalf_w)], bbuf, sem),
        )

    def out_desc(rl, h, obuf, sem):
        row = wid * nrows_w + rl
        xoff = row * row_w + h * half_w
        return pltpu.make_async_copy(obuf, out_hbm.at[pl.ds(xoff, half_w)], sem)

    def compute(ibuf, sbuf, bbuf, obuf):
        def do_d(d, carry):
            s = sbuf[pl.ds(d * _L, _L)]
            b = bbuf[pl.ds(d * _L, _L)]
            for tt in range(t // _L):
                sl = pl.ds(d * t + tt * _L, _L)
                obuf[sl] = ibuf[sl] * s + b
            return carry

        lax.fori_loop(0, hd, do_d, 0)

    for d in in_descs(0, 0, in0, s0, b0, sin0):
        d.start()
    for d in in_descs(0, 1, in1, s1, b1, sin1):
        d.start()

    def half(j, h, ibuf, sbuf, bbuf, obuf, sem_in, sem_out):
        for d in in_descs(j, h, ibuf, sbuf, bbuf, sem_in):
            d.wait()

        @pl.when(j > 0)
        def _():
            out_desc(j - 1, h, obuf, sem_out).wait()

        compute(ibuf, sbuf, bbuf, obuf)
        out_desc(j, h, obuf, sem_out).start()

        @pl.when(j + 1 < nrows_w)
        def _():
            for d in in_descs(j + 1, h, ibuf, sbuf, bbuf, sem_in):
                d.start()

    def body(j, carry):
        half(j, 0, in0, s0, b0, out0, sin0, sout0)
        half(j, 1, in1, s1, b1, out1, sin1, sout1)
        return carry

    lax.fori_loop(0, nrows_w, body, 0)
    out_desc(nrows_w - 1, 0, out0, sout0).wait()
    out_desc(nrows_w - 1, 1, out1, sout1).wait()


def _sc_kernel(x, W_scale, W_bias):
    rows, dim, t = x.shape
    nw = 32
    nrows_w = rows // nw
    assert nrows_w == N_PEAKS_
    xf = x.reshape(rows * dim * t)
    ws16 = jnp.repeat(W_scale.reshape(N_PEAKS_, dim, 1), _L, axis=2).reshape(-1)
    wb16 = jnp.repeat(W_bias.reshape(N_PEAKS_, dim, 1), _L, axis=2).reshape(-1)
    mesh = plsc.VectorSubcoreMesh(core_axis_name="c", subcore_axis_name="s")
    half_w = dim * t // 2
    whalf_w = dim * _L // 2
    f = pl.kernel(
        functools.partial(_sc_body, nrows_w, dim, t),
        out_type=jax.ShapeDtypeStruct((rows * dim * t,), x.dtype),
        mesh=mesh,
        scratch_types=[
            pltpu.VMEM((half_w,), jnp.float32),
            pltpu.VMEM((half_w,), jnp.float32),
            pltpu.VMEM((half_w,), jnp.float32),
            pltpu.VMEM((half_w,), jnp.float32),
            pltpu.VMEM((whalf_w,), jnp.float32),
            pltpu.VMEM((whalf_w,), jnp.float32),
            pltpu.VMEM((whalf_w,), jnp.float32),
            pltpu.VMEM((whalf_w,), jnp.float32),
            pltpu.SemaphoreType.DMA,
            pltpu.SemaphoreType.DMA,
            pltpu.SemaphoreType.DMA,
            pltpu.SemaphoreType.DMA,
        ],
    )
    out = f(xf, ws16, wb16)
    return out.reshape(x.shape)


def kernel(x, queries, W_scale, W_bias):
    del queries
    return _sc_kernel(x, W_scale, W_bias)
    rows, dim, t = x.shape
    R = 64
    grid = (rows // R,)

    out = pl.pallas_call(
        _cond_body,
        grid=grid,
        in_specs=[
            pl.BlockSpec((R, dim, t), lambda i: (i, 0, 0)),
            pl.BlockSpec((N_PEAKS_, dim), lambda i: (0, 0)),
            pl.BlockSpec((N_PEAKS_, dim), lambda i: (0, 0)),
        ],
        out_specs=pl.BlockSpec((R, dim, t), lambda i: (i, 0, 0)),
        out_shape=jax.ShapeDtypeStruct(x.shape, x.dtype),
        compiler_params=pltpu.CompilerParams(
            dimension_semantics=("parallel",),
        ),
    )(x, W_scale, W_bias)
    return out


# speedup vs baseline: 1.4789x; 1.4789x over previous
import functools

import jax
import jax.numpy as jnp
from jax import lax
from jax.experimental import pallas as pl
from jax.experimental.pallas import tpu as pltpu
from jax.experimental.pallas import tpu_sc as plsc

N_PEAKS_ = 64
DIM_ = 128


def _cond_body(x_ref, s_ref, b_ref, o_ref):
    s = s_ref[...][:, :, None]
    b = b_ref[...][:, :, None]
    o_ref[...] = x_ref[...] * s + b


_L = 16


def _sc_body(nrows_w, dim, t, x_hbm, ws16_hbm, wb16_hbm, out_hbm,
             s16_v, b16_v, row_v, sem):
    nc = 2
    wid = lax.axis_index("s") * nc + lax.axis_index("c")
    row_w = dim * t
    wrow_w = dim * _L

    def do_row(k, carry):
        row = wid * nrows_w + k
        base = row * row_w
        woff = k * wrow_w
        c1 = pltpu.make_async_copy(x_hbm.at[pl.ds(base, row_w)], row_v, sem)
        c2 = pltpu.make_async_copy(ws16_hbm.at[pl.ds(woff, wrow_w)], s16_v, sem)
        c3 = pltpu.make_async_copy(wb16_hbm.at[pl.ds(woff, wrow_w)], b16_v, sem)
        c1.start()
        c2.start()
        c3.start()
        c1.wait()
        c2.wait()
        c3.wait()

        def do_d(d2, carry2):
            for u in range(2):
                d = d2 * 2 + u
                s = s16_v[pl.ds(d * _L, _L)]
                b = b16_v[pl.ds(d * _L, _L)]
                for tt in range(t // _L):
                    sl = pl.ds(d * t + tt * _L, _L)
                    row_v[sl] = row_v[sl] * s + b
            return carry2

        lax.fori_loop(0, dim // 2, do_d, 0)
        pltpu.sync_copy(row_v, out_hbm.at[pl.ds(base, row_w)])
        return carry

    lax.fori_loop(0, nrows_w, do_row, 0)


def _sc_kernel(x, W_scale, W_bias):
    rows, dim, t = x.shape
    nw = 32
    nrows_w = rows // nw
    assert nrows_w == N_PEAKS_
    xf = x.reshape(rows * dim * t)
    ws16 = jnp.repeat(W_scale.reshape(N_PEAKS_, dim, 1), _L, axis=2).reshape(-1)
    wb16 = jnp.repeat(W_bias.reshape(N_PEAKS_, dim, 1), _L, axis=2).reshape(-1)
    mesh = plsc.VectorSubcoreMesh(core_axis_name="c", subcore_axis_name="s")
    f = pl.kernel(
        functools.partial(_sc_body, nrows_w, dim, t),
        out_type=jax.ShapeDtypeStruct((rows * dim * t,), x.dtype),
        mesh=mesh,
        scratch_types=[
            pltpu.VMEM((dim * _L,), jnp.float32),
            pltpu.VMEM((dim * _L,), jnp.float32),
            pltpu.VMEM((dim * t,), jnp.float32),
            pltpu.SemaphoreType.DMA,
        ],
    )
    out = f(xf, ws16, wb16)
    return out.reshape(x.shape)


def kernel(x, queries, W_scale, W_bias):
    del queries
    return _sc_kernel(x, W_scale, W_bias)
    rows, dim, t = x.shape
    R = 64
    grid = (rows // R,)

    out = pl.pallas_call(
        _cond_body,
        grid=grid,
        in_specs=[
            pl.BlockSpec((R, dim, t), lambda i: (i, 0, 0)),
            pl.BlockSpec((N_PEAKS_, dim), lambda i: (0, 0)),
            pl.BlockSpec((N_PEAKS_, dim), lambda i: (0, 0)),
        ],
        out_specs=pl.BlockSpec((R, dim, t), lambda i: (i, 0, 0)),
        out_shape=jax.ShapeDtypeStruct(x.shape, x.dtype),
        compiler_params=pltpu.CompilerParams(
            dimension_semantics=("parallel",),
        ),
    )(x, W_scale, W_bias)
    return out


# speedup vs baseline: 3.1612x; 2.1376x over previous
import functools

import jax
import jax.numpy as jnp
from jax import lax
from jax.experimental import pallas as pl
from jax.experimental.pallas import tpu as pltpu
from jax.experimental.pallas import tpu_sc as plsc

N_PEAKS_ = 64
DIM_ = 128


def _cond_body(x_ref, s_ref, b_ref, o_ref):
    s = s_ref[...][:, :, None]
    b = b_ref[...][:, :, None]
    o_ref[...] = x_ref[...] * s + b


_L = 16


def _sc_body(nrows_w, dim, t, x_hbm, ws16_hbm, wb16_hbm, out_hbm,
             s16_v, b16_v, row_v, sem):
    nc = 2
    wid = lax.axis_index("s") * nc + lax.axis_index("c")
    row_w = dim * t
    wrow_w = dim * _L

    def do_row(k, carry):
        row = wid * nrows_w + k
        c1 = pltpu.make_async_copy(x_hbm.at[row], row_v, sem)
        c2 = pltpu.make_async_copy(ws16_hbm.at[k], s16_v, sem)
        c3 = pltpu.make_async_copy(wb16_hbm.at[k], b16_v, sem)
        c1.start()
        c2.start()
        c3.start()
        c1.wait()
        c2.wait()
        c3.wait()

        def do_d(d2, carry2):
            for u in range(2):
                d = d2 * 2 + u
                s = s16_v[pl.ds(d * _L, _L)]
                b = b16_v[pl.ds(d * _L, _L)]
                for tt in range(t // _L):
                    sl = pl.ds(tt * _L, _L)
                    row_v[d, sl] = row_v[d, sl] * s + b
            return carry2

        lax.fori_loop(0, dim // 2, do_d, 0)
        pltpu.sync_copy(row_v, out_hbm.at[row])
        return carry

    lax.fori_loop(0, nrows_w, do_row, 0)


def _sc_kernel(x, W_scale, W_bias):
    rows, dim, t = x.shape
    nw = 32
    nrows_w = rows // nw
    assert nrows_w == N_PEAKS_
    ws16 = jnp.repeat(W_scale.reshape(N_PEAKS_, dim, 1), _L, axis=2).reshape(
        N_PEAKS_, dim * _L)
    wb16 = jnp.repeat(W_bias.reshape(N_PEAKS_, dim, 1), _L, axis=2).reshape(
        N_PEAKS_, dim * _L)
    mesh = plsc.VectorSubcoreMesh(core_axis_name="c", subcore_axis_name="s")
    f = pl.kernel(
        functools.partial(_sc_body, nrows_w, dim, t),
        out_type=jax.ShapeDtypeStruct(x.shape, x.dtype),
        mesh=mesh,
        scratch_types=[
            pltpu.VMEM((dim * _L,), jnp.float32),
            pltpu.VMEM((dim * _L,), jnp.float32),
            pltpu.VMEM((dim, t), jnp.float32),
            pltpu.SemaphoreType.DMA,
        ],
    )
    return f(x, ws16, wb16)


def kernel(x, queries, W_scale, W_bias):
    del queries
    return _sc_kernel(x, W_scale, W_bias)
    rows, dim, t = x.shape
    R = 64
    grid = (rows // R,)

    out = pl.pallas_call(
        _cond_body,
        grid=grid,
        in_specs=[
            pl.BlockSpec((R, dim, t), lambda i: (i, 0, 0)),
            pl.BlockSpec((N_PEAKS_, dim), lambda i: (0, 0)),
            pl.BlockSpec((N_PEAKS_, dim), lambda i: (0, 0)),
        ],
        out_specs=pl.BlockSpec((R, dim, t), lambda i: (i, 0, 0)),
        out_shape=jax.ShapeDtypeStruct(x.shape, x.dtype),
        compiler_params=pltpu.CompilerParams(
            dimension_semantics=("parallel",),
        ),
    )(x, W_scale, W_bias)
    return out


# speedup vs baseline: 5.2727x; 1.6679x over previous
import functools

import jax
import jax.numpy as jnp
from jax import lax
from jax.experimental import pallas as pl
from jax.experimental.pallas import tpu as pltpu
from jax.experimental.pallas import tpu_sc as plsc

N_PEAKS_ = 64
DIM_ = 128


def _cond_body(x_ref, s_ref, b_ref, o_ref):
    s = s_ref[...][:, :, None]
    b = b_ref[...][:, :, None]
    o_ref[...] = x_ref[...] * s + b


_L = 16


def _sc_body(nrows_w, dim, t, x_hbm, ws16_hbm, wb16_hbm, out_hbm,
             in0, in1, in2, s0, s1, s2, b0, b1, b2,
             sin0, sin1, sin2, sout0, sout1, sout2):
    nc = 2
    wid = lax.axis_index("s") * nc + lax.axis_index("c")
    base_row = wid * nrows_w
    bufs = (
        (in0, s0, b0, sin0, sout0),
        (in1, s1, b1, sin1, sout1),
        (in2, s2, b2, sin2, sout2),
    )
    last = nrows_w - 1

    def in_cps(k, bi):
        ibuf, sbuf, bbuf, si, _ = bufs[bi]
        return (
            pltpu.make_async_copy(x_hbm.at[base_row + k], ibuf, si),
            pltpu.make_async_copy(ws16_hbm.at[k], sbuf, si),
            pltpu.make_async_copy(wb16_hbm.at[k], bbuf, si),
        )

    def out_cp(k, bi):
        ibuf = bufs[bi][0]
        so = bufs[bi][4]
        return pltpu.make_async_copy(ibuf, out_hbm.at[base_row + k], so)

    def compute(bi):
        ibuf, sbuf, bbuf, _, _ = bufs[bi]

        def do_d(d2, carry2):
            for u in range(2):
                d = d2 * 2 + u
                s = sbuf[pl.ds(d * _L, _L)]
                b = bbuf[pl.ds(d * _L, _L)]
                for tt in range(t // _L):
                    sl = pl.ds(tt * _L, _L)
                    ibuf[d, sl] = ibuf[d, sl] * s + b
            return carry2

        lax.fori_loop(0, dim // 2, do_d, 0)

    def row_step(k, bi, prefetch):
        for c in in_cps(k, bi):
            c.wait()
        compute(bi)
        out_cp(k, bi).start()
        if prefetch:
            nbi = (bi + 2) % 3

            @pl.when((k >= 1) & (k <= last - 2))
            def _():
                out_cp(k - 1, nbi).wait()

            @pl.when(k <= last - 2)
            def _():
                for c in in_cps(k + 2, nbi):
                    c.start()

    for c in in_cps(0, 0):
        c.start()
    for c in in_cps(1, 1):
        c.start()

    def body(j, carry):
        a = 3 * j
        row_step(a, 0, True)
        row_step(a + 1, 1, True)
        row_step(a + 2, 2, True)
        return carry

    lax.fori_loop(0, nrows_w // 3, body, 0)
    row_step(last, 0, False)
    out_cp(last - 2, 1).wait()
    out_cp(last - 1, 2).wait()
    out_cp(last, 0).wait()


def _sc_kernel(x, W_scale, W_bias):
    rows, dim, t = x.shape
    nw = 32
    nrows_w = rows // nw
    assert nrows_w == N_PEAKS_
    ws16 = jnp.repeat(W_scale.reshape(N_PEAKS_, dim, 1), _L, axis=2).reshape(
        N_PEAKS_, dim * _L)
    wb16 = jnp.repeat(W_bias.reshape(N_PEAKS_, dim, 1), _L, axis=2).reshape(
        N_PEAKS_, dim * _L)
    mesh = plsc.VectorSubcoreMesh(core_axis_name="c", subcore_axis_name="s")
    f = pl.kernel(
        functools.partial(_sc_body, nrows_w, dim, t),
        out_type=jax.ShapeDtypeStruct(x.shape, x.dtype),
        mesh=mesh,
        scratch_types=(
            [pltpu.VMEM((dim, t), jnp.float32)] * 3
            + [pltpu.VMEM((dim * _L,), jnp.float32)] * 6
            + [pltpu.SemaphoreType.DMA] * 6
        ),
    )
    return f(x, ws16, wb16)


def kernel(x, queries, W_scale, W_bias):
    del queries
    return _sc_kernel(x, W_scale, W_bias)
    rows, dim, t = x.shape
    R = 64
    grid = (rows // R,)

    out = pl.pallas_call(
        _cond_body,
        grid=grid,
        in_specs=[
            pl.BlockSpec((R, dim, t), lambda i: (i, 0, 0)),
            pl.BlockSpec((N_PEAKS_, dim), lambda i: (0, 0)),
            pl.BlockSpec((N_PEAKS_, dim), lambda i: (0, 0)),
        ],
        out_specs=pl.BlockSpec((R, dim, t), lambda i: (i, 0, 0)),
        out_shape=jax.ShapeDtypeStruct(x.shape, x.dtype),
        compiler_params=pltpu.CompilerParams(
            dimension_semantics=("parallel",),
        ),
    )(x, W_scale, W_bias)
    return out
